# Initial kernel scaffold; baseline (speedup 1.0000x reference)
import jax
import jax.numpy as jnp
from jax.experimental import pallas as pl


def kernel(selected_ab, x_ag, edge_index, W1, a_src1, a_dst1, b1, W2, a_src2, a_dst2, b2, bn2_g, bn2_b, bn2_m, bn2_v, ag_g, ag_b, ag_m, ag_v, fc_w, fc_b, agfc_w, agfc_b):
    raise NotImplementedError("write your pallas kernel here")


# speedup vs baseline: 21.1451x; 21.1451x over previous
import functools

import jax
import jax.numpy as jnp
from jax import lax
from jax.experimental import pallas as pl
from jax.experimental.pallas import tpu as pltpu
from jax.experimental.pallas import tpu_sc as plsc

D = 128
LANES = 16
CHUNK = 128
NT = 32
SUB_ROWS = 640


def _mm1_body(x_ref, w_ref, asrc_ref, adst_ref, h_ref, scal_ref):
    h = jnp.dot(x_ref[...], w_ref[...], preferred_element_type=jnp.float32)
    h_ref[...] = h
    scal_ref[0, :] = jnp.sum(h * asrc_ref[...], axis=1)
    scal_ref[1, :] = jnp.sum(h * adst_ref[...], axis=1)


def _combine_mm_body(acc_ref, den_ref, b_ref, w_ref, asrc_ref, adst_ref,
                     h_ref, scal_ref):
    den = jnp.sum(den_ref[...], axis=0) + 1e-16
    x = (acc_ref[0] + acc_ref[1]) / den[:, None] + b_ref[...]
    x = jnp.maximum(x, 0.0)
    h = jnp.dot(x, w_ref[...], preferred_element_type=jnp.float32)
    h_ref[...] = h
    scal_ref[0, :] = jnp.sum(h * asrc_ref[...], axis=1)
    scal_ref[1, :] = jnp.sum(h * adst_ref[...], axis=1)


def _final_body(acc_ref, den_ref, b_ref, ab_ref, ag_ref,
                bn2g_ref, bn2b_ref, bn2m_ref, bn2v_ref,
                agg_ref, agb_ref, agm_ref, agv_ref,
                fcw_ref, fcb_ref, agfcw_ref, agfcb_ref,
                oab_ref, oag_ref):
    nab = ab_ref.shape[0]
    nag = ag_ref.shape[0]
    den = jnp.sum(den_ref[...], axis=0) + 1e-16
    x2 = (acc_ref[0] + acc_ref[1]) / den[:, None] + b_ref[...]
    xab = jnp.concatenate([x2[:nab], ab_ref[...]], axis=1)
    xab = (xab - bn2m_ref[...]) / jnp.sqrt(bn2v_ref[...] + 1e-5) * bn2g_ref[...] + bn2b_ref[...]
    xab = jnp.maximum(xab, 0.0)
    oab_ref[...] = jnp.dot(xab, fcw_ref[...], preferred_element_type=jnp.float32) + fcb_ref[0, 0]
    xg = jnp.concatenate([x2[nab:nab + nag], ag_ref[...]], axis=1)
    xg = (xg - agm_ref[...]) / jnp.sqrt(agv_ref[...] + 1e-5) * agg_ref[...] + agb_ref[...]
    xg = jnp.maximum(xg, 0.0)
    oag_ref[...] = jnp.dot(xg, agfcw_ref[...], preferred_element_type=jnp.float32) + agfcb_ref[0, 0]


def _make_sc_edge_kernel(n_pad, n_chunks):
    mesh = plsc.VectorSubcoreMesh(core_axis_name="c", subcore_axis_name="s")

    @functools.partial(
        pl.kernel,
        mesh=mesh,
        compiler_params=pltpu.CompilerParams(needs_layout_passes=False),
        out_type=[
            jax.ShapeDtypeStruct((2, n_pad, D), jnp.float32),
            jax.ShapeDtypeStruct((NT, n_pad), jnp.float32),
        ],
        scratch_types=[
            pltpu.VMEM((n_pad,), jnp.float32),
            pltpu.VMEM((n_pad,), jnp.float32),
            pltpu.VMEM((n_pad,), jnp.float32),
            pltpu.VMEM((CHUNK,), jnp.int32),
            pltpu.VMEM((CHUNK,), jnp.int32),
            pltpu.VMEM((CHUNK, D), jnp.float32),
            pltpu.VMEM((CHUNK,), jnp.float32),
            pltpu.VMEM_SHARED((n_pad, D), jnp.float32),
            pltpu.SemaphoreType.DMA,
        ],
    )
    def sc_edge(h_hbm, asrc_hbm, adst_hbm, src_hbm, dst_hbm,
                acc_out, den_out,
                asrc_t, adst_t, denom_t, src_t, dst_t, rows, ex_t,
                acc_sh, sem):
        c = lax.axis_index("c")
        s = lax.axis_index("s")
        wid = s * 2 + c

        pltpu.sync_copy(asrc_hbm, asrc_t)
        pltpu.sync_copy(adst_hbm, adst_t)

        zero16 = jnp.zeros((LANES,), jnp.float32)

        def zden(i, carry):
            denom_t[pl.ds(i * LANES, LANES)] = zero16
            return carry
        lax.fori_loop(0, n_pad // LANES, zden, 0)

        def zrow(i, carry):
            for j in range(D // LANES):
                rows[i, pl.ds(j * LANES, LANES)] = zero16
            return carry
        lax.fori_loop(0, CHUNK, zrow, 0)

        for t in range(SUB_ROWS // CHUNK):
            pltpu.sync_copy(rows, acc_sh.at[pl.ds(s * SUB_ROWS + t * CHUNK, CHUNK)])
        plsc.subcore_barrier()

        def chunk_body(k, carry):
            pltpu.sync_copy(src_hbm.at[wid, k], src_t)
            pltpu.sync_copy(dst_hbm.at[wid, k], dst_t)
            pltpu.async_copy(h_hbm.at[src_t], rows, sem).wait()

            def grp(g, c2):
                sidx = src_t[pl.ds(g * LANES, LANES)]
                didx = dst_t[pl.ds(g * LANES, LANES)]
                a = plsc.load_gather(asrc_t, [sidx]) + plsc.load_gather(adst_t, [didx])
                al = jnp.where(a >= 0.0, a, a * 0.2)
                ex = jnp.exp(al)
                plsc.addupdate_scatter(denom_t, [didx], ex)
                ex_t[pl.ds(g * LANES, LANES)] = ex
                return c2
            lax.fori_loop(0, CHUNK // LANES, grp, 0)

            def scale(e, c3):
                exb = plsc.load_gather(ex_t, [jnp.full((LANES,), e, jnp.int32)])
                for j in range(D // LANES):
                    rows[e, pl.ds(j * LANES, LANES)] = rows[e, pl.ds(j * LANES, LANES)] * exb
                return c3
            lax.fori_loop(0, CHUNK, scale, 0)

            pltpu.sync_copy(rows, acc_sh.at[dst_t], add=True)
            return carry
        lax.fori_loop(0, n_chunks, chunk_body, 0)

        pltpu.sync_copy(denom_t, den_out.at[wid])
        plsc.subcore_barrier()
        for t in range(SUB_ROWS // CHUNK):
            off = s * SUB_ROWS + t * CHUNK
            pltpu.sync_copy(acc_sh.at[pl.ds(off, CHUNK)],
                            acc_out.at[c, pl.ds(off, CHUNK)])

    return sc_edge


def kernel(selected_ab, x_ag, edge_index, W1, a_src1, a_dst1, b1,
           W2, a_src2, a_dst2, b2,
           bn2_g, bn2_b, bn2_m, bn2_v, ag_g, ag_b, ag_m, ag_v,
           fc_w, fc_b, agfc_w, agfc_b):
    nab = selected_ab.shape[0]
    nag = x_ag.shape[0]
    n = nab + nag
    e_tot = edge_index.shape[1] + n
    n_chunks = -(-e_tot // (NT * CHUNK))
    ept = n_chunks * CHUNK
    pad_e = NT * ept - e_tot
    n_pad = -(-n // SUB_ROWS) * SUB_ROWS

    x = jnp.concatenate(
        [selected_ab, x_ag, jnp.zeros((n_pad - n, D), jnp.float32)], axis=0)
    loops = jnp.arange(n, dtype=jnp.int32)
    src = jnp.concatenate(
        [edge_index[0], loops, jnp.zeros((pad_e,), jnp.int32)])
    dst = jnp.concatenate(
        [edge_index[1], loops, jnp.full((pad_e,), n, jnp.int32)])
    src3 = src.reshape(NT, n_chunks, CHUNK)
    dst3 = dst.reshape(NT, n_chunks, CHUNK)

    mm1 = pl.pallas_call(
        _mm1_body,
        out_shape=[jax.ShapeDtypeStruct((n_pad, D), jnp.float32),
                   jax.ShapeDtypeStruct((2, n_pad), jnp.float32)],
    )
    h1, scal1 = mm1(x, W1, a_src1.reshape(1, D), a_dst1.reshape(1, D))

    sc_edge = _make_sc_edge_kernel(n_pad, n_chunks)
    acc1, den1 = sc_edge(h1, scal1[0], scal1[1], src3, dst3)

    mm2 = pl.pallas_call(
        _combine_mm_body,
        out_shape=[jax.ShapeDtypeStruct((n_pad, D), jnp.float32),
                   jax.ShapeDtypeStruct((2, n_pad), jnp.float32)],
    )
    h2, scal2 = mm2(acc1, den1, b1.reshape(1, D), W2,
                    a_src2.reshape(1, D), a_dst2.reshape(1, D))

    acc2, den2 = sc_edge(h2, scal2[0], scal2[1], src3, dst3)

    fin = pl.pallas_call(
        _final_body,
        out_shape=[jax.ShapeDtypeStruct((nab, 1), jnp.float32),
                   jax.ShapeDtypeStruct((nag, 1), jnp.float32)],
    )
    yab, yg = fin(acc2, den2, b2.reshape(1, D), selected_ab, x_ag,
                  bn2_g.reshape(1, 2 * D), bn2_b.reshape(1, 2 * D),
                  bn2_m.reshape(1, 2 * D), bn2_v.reshape(1, 2 * D),
                  ag_g.reshape(1, 2 * D), ag_b.reshape(1, 2 * D),
                  ag_m.reshape(1, 2 * D), ag_v.reshape(1, 2 * D),
                  fc_w, fc_b.reshape(1, 1), agfc_w, agfc_b.reshape(1, 1))
    return (yab.reshape(-1), yg.reshape(-1))
